# baseline (speedup 1.0000x reference)
import functools

import jax
import jax.numpy as jnp
from jax import lax
from jax.experimental import pallas as pl
from jax.experimental.pallas import tpu as pltpu
from jax.experimental.pallas import tpu_sc as plsc

B = 16384
D = 32
L = 16
NW = 32
BPW = B // NW
NG = BPW // L


NSEM = 8


def _gmf_body(uidx_hbm, iidx_hbm, utab, itab, w_hbm, b_hbm,
              out_hbm, uidx_v, iidx_v, ublk, iblk, w_v, b_v, tbuf, outbuf,
              *sems):
    wid = lax.axis_index("c") * 16 + lax.axis_index("s")
    base = wid * BPW

    pltpu.sync_copy(uidx_hbm.at[pl.ds(base, BPW)], uidx_v)
    pltpu.sync_copy(iidx_hbm.at[pl.ds(base, BPW)], iidx_v)
    pltpu.sync_copy(w_hbm, w_v)
    pltpu.sync_copy(b_hbm, b_v)

    w0 = w_v[0, :]
    w1 = w_v[1, :]
    b_vec = b_v[...]
    lane = lax.iota(jnp.int32, L)
    ia = lane // 8
    ir = lane - (lane // 8) * 8
    lane_scaled = lane * L

    def group(g, carry):
        jbase = g * L
        jv_u = uidx_v[pl.ds(jbase, L)]
        jv_i = iidx_v[pl.ds(jbase, L)]
        copies = []
        for i in range(L):
            ju = jv_u[i]
            ji = jv_i[i]
            j8u = (ju // 8) * 8
            j8i = (ji // 8) * 8
            for a in range(4):
                copies.append(pltpu.async_copy(
                    utab.at[a, :, pl.ds(j8u, 8)],
                    ublk.at[a, :, pl.ds(i * 8, 8)], sems[(2 * a) % NSEM]))
                copies.append(pltpu.async_copy(
                    itab.at[a, :, pl.ds(j8i, 8)],
                    iblk.at[a, :, pl.ds(i * 8, 8)], sems[(2 * a + 1) % NSEM]))
        for c in copies:
            c.wait()
        for i in range(L):
            ju = jv_u[i]
            ji = jv_i[i]
            cu = jnp.full((L,), i * 8 + (ju - (ju // 8) * 8), jnp.int32)
            ci = jnp.full((L,), i * 8 + (ji - (ji // 8) * 8), jnp.int32)
            u_lo = plsc.load_gather(ublk, [ia, ir, cu])
            u_hi = plsc.load_gather(ublk, [ia + 2, ir, cu])
            v_lo = plsc.load_gather(iblk, [ia, ir, ci])
            v_hi = plsc.load_gather(iblk, [ia + 2, ir, ci])
            ps = u_lo * v_lo * w0 + u_hi * v_hi * w1
            plsc.store_scatter(tbuf, [lane_scaled + i], ps)
        parts = [tbuf[pl.ds(k * L, L)] for k in range(L)]
        while len(parts) > 1:
            parts = [parts[k] + parts[k + 1] for k in range(0, len(parts), 2)]
        logit = parts[0] + b_vec
        outbuf[pl.ds(jbase, L)] = 1.0 / (1.0 + jnp.exp(-logit))
        return carry

    lax.fori_loop(0, NG, group, None)

    pltpu.sync_copy(outbuf, out_hbm.at[pl.ds(base, BPW)])


@jax.jit
def _gmf(uidx, iidx, utab3, itab3, w2, b16):
    mesh = plsc.VectorSubcoreMesh(core_axis_name="c", subcore_axis_name="s")
    run = functools.partial(
        pl.kernel,
        out_type=jax.ShapeDtypeStruct((B,), jnp.float32),
        mesh=mesh,
        scratch_types=[
            pltpu.VMEM((BPW,), jnp.int32),
            pltpu.VMEM((BPW,), jnp.int32),
            pltpu.VMEM((4, 8, L * 8), jnp.float32),
            pltpu.VMEM((4, 8, L * 8), jnp.float32),
            pltpu.VMEM((2, L), jnp.float32),
            pltpu.VMEM((L,), jnp.float32),
            pltpu.VMEM((L * L,), jnp.float32),
            pltpu.VMEM((BPW,), jnp.float32),
        ] + [pltpu.SemaphoreType.DMA] * NSEM,
        compiler_params=pltpu.CompilerParams(
            needs_layout_passes=False, use_tc_tiling_on_sc=True),
    )(_gmf_body)
    return run(uidx, iidx, utab3, itab3, w2, b16)


def kernel(input, user_table, item_table, W, b):
    uidx = input[:, 0]
    iidx = input[:, 1]
    utab3 = user_table.T.reshape(4, 8, user_table.shape[0])
    itab3 = item_table.T.reshape(4, 8, item_table.shape[0])
    w2 = W.reshape(2, L)
    b16 = jnp.broadcast_to(b, (L,))
    return _gmf(uidx, iidx, utab3, itab3, w2, b16)


# speedup vs baseline: 1.1249x; 1.1249x over previous
import functools

import jax
import jax.numpy as jnp
from jax import lax
from jax.experimental import pallas as pl
from jax.experimental.pallas import tpu as pltpu
from jax.experimental.pallas import tpu_sc as plsc

B = 16384
D = 32
L = 16
NW = 32
BPW = B // NW
C = 64
NCHK = BPW // C
NGC = C // L
CW = C * 8


def _gmf_body(uidx_hbm, iidx_hbm, utab, itab, w_hbm, b_hbm,
              out_hbm, uidx_v, iidx_v, ublk, vblk, w_v, b_v, tbuf, outbuf,
              sem0, sem1):
    wid = lax.axis_index("c") * 16 + lax.axis_index("s")
    base = wid * BPW

    pltpu.sync_copy(uidx_hbm.at[pl.ds(base, BPW)], uidx_v)
    pltpu.sync_copy(iidx_hbm.at[pl.ds(base, BPW)], iidx_v)
    pltpu.sync_copy(w_hbm, w_v)
    pltpu.sync_copy(b_hbm, b_v)

    w0 = w_v[0, :]
    w1 = w_v[1, :]
    b_vec = b_v[...]
    lane = lax.iota(jnp.int32, L)
    ia = lane // 8
    ir = lane - (lane // 8) * 8
    lane_scaled = lane * L

    def fire_chunk(c, boff, sem):
        def body(g, carry):
            jbase = c * C + g * L
            jv_u = uidx_v[pl.ds(jbase, L)]
            jv_i = iidx_v[pl.ds(jbase, L)]
            col = boff + g * (L * 8)
            for i in range(L):
                ju = jv_u[i]
                ji = jv_i[i]
                j8u = (ju // 8) * 8
                j8i = (ji // 8) * 8
                pltpu.async_copy(utab.at[:, :, pl.ds(j8u, 8)],
                                 ublk.at[:, :, pl.ds(col + i * 8, 8)], sem)
                pltpu.async_copy(itab.at[:, :, pl.ds(j8i, 8)],
                                 vblk.at[:, :, pl.ds(col + i * 8, 8)], sem)
            return carry
        lax.fori_loop(0, NGC, body, None)

    def drain_chunk(boff, sem):
        pltpu.make_async_copy(utab.at[:, :, pl.ds(0, CW)],
                              ublk.at[:, :, pl.ds(boff, CW)], sem).wait()
        pltpu.make_async_copy(itab.at[:, :, pl.ds(0, CW)],
                              vblk.at[:, :, pl.ds(boff, CW)], sem).wait()

    def compute_chunk(c, boff):
        def body(g, carry):
            jbase = c * C + g * L
            jv_u = uidx_v[pl.ds(jbase, L)]
            jv_i = iidx_v[pl.ds(jbase, L)]
            col = boff + g * (L * 8)
            for i in range(L):
                ju = jv_u[i]
                ji = jv_i[i]
                cu = jnp.full((L,), col + i * 8 + (ju - (ju // 8) * 8),
                              jnp.int32)
                ci = jnp.full((L,), col + i * 8 + (ji - (ji // 8) * 8),
                              jnp.int32)
                u_lo = plsc.load_gather(ublk, [ia, ir, cu])
                u_hi = plsc.load_gather(ublk, [ia + 2, ir, cu])
                v_lo = plsc.load_gather(vblk, [ia, ir, ci])
                v_hi = plsc.load_gather(vblk, [ia + 2, ir, ci])
                ps = u_lo * v_lo * w0 + u_hi * v_hi * w1
                plsc.store_scatter(tbuf, [lane_scaled + i], ps)
            parts = [tbuf[pl.ds(k * L, L)] for k in range(L)]
            while len(parts) > 1:
                parts = [parts[k] + parts[k + 1]
                         for k in range(0, len(parts), 2)]
            logit = parts[0] + b_vec
            outbuf[pl.ds(jbase, L)] = 1.0 / (1.0 + jnp.exp(-logit))
            return carry
        lax.fori_loop(0, NGC, body, None)

    fire_chunk(0, 0, sem0)

    def step(c, carry):
        par = c - (c // 2) * 2
        nxt = c + 1
        npar = nxt - (nxt // 2) * 2
        cur_off = par * CW

        @pl.when(jnp.logical_and(nxt < NCHK, npar == 0))
        def _():
            fire_chunk(nxt, 0, sem0)

        @pl.when(jnp.logical_and(nxt < NCHK, npar == 1))
        def _():
            fire_chunk(nxt, CW, sem1)

        @pl.when(par == 0)
        def _():
            drain_chunk(0, sem0)

        @pl.when(par == 1)
        def _():
            drain_chunk(CW, sem1)

        compute_chunk(c, cur_off)
        return carry

    lax.fori_loop(0, NCHK, step, None)

    pltpu.sync_copy(outbuf, out_hbm.at[pl.ds(base, BPW)])


@jax.jit
def _gmf(uidx, iidx, utab3, itab3, w2, b16):
    mesh = plsc.VectorSubcoreMesh(core_axis_name="c", subcore_axis_name="s")
    run = functools.partial(
        pl.kernel,
        out_type=jax.ShapeDtypeStruct((B,), jnp.float32),
        mesh=mesh,
        scratch_types=[
            pltpu.VMEM((BPW,), jnp.int32),
            pltpu.VMEM((BPW,), jnp.int32),
            pltpu.VMEM((4, 8, 2 * CW), jnp.float32),
            pltpu.VMEM((4, 8, 2 * CW), jnp.float32),
            pltpu.VMEM((2, L), jnp.float32),
            pltpu.VMEM((L,), jnp.float32),
            pltpu.VMEM((L * L,), jnp.float32),
            pltpu.VMEM((BPW,), jnp.float32),
            pltpu.SemaphoreType.DMA,
            pltpu.SemaphoreType.DMA,
        ],
        compiler_params=pltpu.CompilerParams(
            needs_layout_passes=False, use_tc_tiling_on_sc=True),
    )(_gmf_body)
    return run(uidx, iidx, utab3, itab3, w2, b16)


def kernel(input, user_table, item_table, W, b):
    uidx = input[:, 0]
    iidx = input[:, 1]
    utab3 = user_table.T.reshape(4, 8, user_table.shape[0])
    itab3 = item_table.T.reshape(4, 8, item_table.shape[0])
    w2 = W.reshape(2, L)
    b16 = jnp.broadcast_to(b, (L,))
    return _gmf(uidx, iidx, utab3, itab3, w2, b16)
